# Initial kernel scaffold; baseline (speedup 1.0000x reference)
import jax
import jax.numpy as jnp
from jax import lax
from jax.experimental import pallas as pl
from jax.experimental.pallas import tpu as pltpu
from jax.experimental.pallas import tpu_sc as plsc

VOCAB = 100000
WORD_DIM = 128
TAG_NUM = 64
TAG_DIM = 32
OUT_DIM = WORD_DIM + TAG_DIM

NC = 2
NS = 16
NW = NC * NS

CHUNK = 128


def _emb_kernel(word_ids_hbm, tag_ids_hbm, word_table_hbm, tag_table_hbm,
                out_hbm, idx_w_v, idx_t_v, rows_w_v, rows_t_v, sem_w, sem_t):
    n_total = word_ids_hbm.shape[0]
    per_w = n_total // NW
    n_chunks = per_w // CHUNK
    wid = lax.axis_index("s") * NC + lax.axis_index("c")
    base = wid * per_w

    def body(i, _):
        off = base + i * CHUNK
        pltpu.sync_copy(word_ids_hbm.at[pl.ds(off, CHUNK)], idx_w_v)
        pltpu.sync_copy(tag_ids_hbm.at[pl.ds(off, CHUNK)], idx_t_v)
        cw = pltpu.async_copy(word_table_hbm.at[idx_w_v], rows_w_v, sem_w)
        ct = pltpu.async_copy(tag_table_hbm.at[idx_t_v], rows_t_v, sem_t)
        cw.wait()
        ct.wait()
        pltpu.sync_copy(rows_w_v, out_hbm.at[pl.ds(off, CHUNK), pl.ds(0, WORD_DIM)])
        pltpu.sync_copy(rows_t_v, out_hbm.at[pl.ds(off, CHUNK), pl.ds(WORD_DIM, TAG_DIM)])
        return ()

    lax.fori_loop(0, n_chunks, body, (), unroll=False)


def kernel(word_ids, tag_ids, word_table, tag_table):
    b, s = word_ids.shape
    n_total = b * s
    wf = word_ids.reshape(n_total).astype(jnp.int32)
    tf = tag_ids.reshape(n_total).astype(jnp.int32)

    run = pl.kernel(
        _emb_kernel,
        out_type=jax.ShapeDtypeStruct((n_total, OUT_DIM), jnp.float32),
        mesh=plsc.VectorSubcoreMesh(core_axis_name="c", subcore_axis_name="s"),
        scratch_types=[
            pltpu.VMEM((CHUNK,), jnp.int32),
            pltpu.VMEM((CHUNK,), jnp.int32),
            pltpu.VMEM((CHUNK, WORD_DIM), jnp.float32),
            pltpu.VMEM((CHUNK, TAG_DIM), jnp.float32),
            pltpu.SemaphoreType.DMA,
            pltpu.SemaphoreType.DMA,
        ],
    )
    out = run(wf, tf, word_table, tag_table)
    return out.reshape(b, s, OUT_DIM)


# speedup vs baseline: 2.4185x; 2.4185x over previous
import jax
import jax.numpy as jnp
from jax import lax
from jax.experimental import pallas as pl
from jax.experimental.pallas import tpu as pltpu
from jax.experimental.pallas import tpu_sc as plsc

VOCAB = 100000
WORD_DIM = 128
TAG_NUM = 64
TAG_DIM = 32
OUT_DIM = WORD_DIM + TAG_DIM

NC = 2
NS = 16
NW = NC * NS

CHUNK = 128


def _emb_kernel(word_ids_hbm, tag_ids_hbm, word_table_hbm, tag_table_hbm,
                out_hbm, idx_w_v, idx_t_v, rows_w_v, rows_t_v, sem_w, sem_t):
    n_total = word_ids_hbm.shape[0]
    per_w = n_total // NW
    n_chunks = per_w // CHUNK
    wid = lax.axis_index("s") * NC + lax.axis_index("c")
    base = wid * per_w

    def body(i, _):
        off = base + i * CHUNK
        pltpu.sync_copy(word_ids_hbm.at[pl.ds(off, CHUNK)], idx_w_v)
        pltpu.sync_copy(tag_ids_hbm.at[pl.ds(off, CHUNK)], idx_t_v)
        cw = pltpu.async_copy(word_table_hbm.at[idx_w_v], rows_w_v, sem_w)
        ct = pltpu.async_copy(tag_table_hbm.at[idx_t_v], rows_t_v, sem_t)
        cw.wait()
        ct.wait()
        pltpu.sync_copy(rows_w_v, out_hbm.at[pl.ds(off, CHUNK), pl.ds(0, WORD_DIM)])
        pltpu.sync_copy(rows_t_v, out_hbm.at[pl.ds(off, CHUNK), pl.ds(WORD_DIM, TAG_DIM)])
        return ()

    lax.fori_loop(0, n_chunks, body, (), unroll=False)


def kernel(word_ids, tag_ids, word_table, tag_table):
    b, s = word_ids.shape
    n_total = b * s
    wf = word_ids.reshape(n_total).astype(jnp.int32)
    tf = tag_ids.reshape(n_total).astype(jnp.int32)

    run = pl.kernel(
        _emb_kernel,
        out_type=jax.ShapeDtypeStruct((n_total, OUT_DIM), jnp.float32),
        mesh=plsc.VectorSubcoreMesh(core_axis_name="c", subcore_axis_name="s"),
        compiler_params=pltpu.CompilerParams(use_tc_tiling_on_sc=False),
        scratch_types=[
            pltpu.VMEM((CHUNK,), jnp.int32),
            pltpu.VMEM((CHUNK,), jnp.int32),
            pltpu.VMEM((CHUNK, WORD_DIM), jnp.float32),
            pltpu.VMEM((CHUNK, TAG_DIM), jnp.float32),
            pltpu.SemaphoreType.DMA,
            pltpu.SemaphoreType.DMA,
        ],
    )
    out = run(wf, tf, word_table, tag_table)
    return out.reshape(b, s, OUT_DIM)


# speedup vs baseline: 6.2582x; 2.5877x over previous
import jax
import jax.numpy as jnp
from jax import lax
from jax.experimental import pallas as pl
from jax.experimental.pallas import tpu as pltpu
from jax.experimental.pallas import tpu_sc as plsc

WORD_DIM = 128
TAG_NUM = 64
TAG_DIM = 32
OUT_DIM = WORD_DIM + TAG_DIM

NC = 2
NS = 16
NW = NC * NS

CHUNK = 128
NBUF = 2


def _emb_kernel(word_ids_hbm, tag_ids_hbm, word_table_hbm, tag_flat_hbm,
                out_hbm, idx_w_v, idx_t_v, tag_v, rows_w_v, rows_t_v,
                sem_g, sem_o):
    n_total = word_ids_hbm.shape[0]
    per_w = n_total // NW
    n_chunks = per_w // CHUNK
    wid = lax.axis_index("s") * NC + lax.axis_index("c")
    base = wid * per_w

    pltpu.sync_copy(word_ids_hbm.at[pl.ds(base, per_w)], idx_w_v)
    pltpu.sync_copy(tag_ids_hbm.at[pl.ds(base, per_w)], idx_t_v)
    pltpu.sync_copy(tag_flat_hbm, tag_v)

    def gather_chunk(i, p):
        pltpu.async_copy(
            word_table_hbm.at[idx_w_v.at[pl.ds(i * CHUNK, CHUNK)]],
            rows_w_v[p], sem_g[p])

    def tag_chunk(i, p):
        def grp(g, _):
            tv = idx_t_v[pl.ds(i * CHUNK + g * 16, 16)]
            offs = tv * TAG_DIM
            row0 = g * 16
            for r in range(16):
                off = offs[r]
                rows_t_v[p][row0 + r, 0:16] = tag_v[pl.ds(off, 16)]
                rows_t_v[p][row0 + r, 16:32] = tag_v[pl.ds(off + 16, 16)]
            return ()
        lax.fori_loop(0, CHUNK // 16, grp, (), unroll=False)

    def out_copies(i, p):
        off = base + i * CHUNK
        cw = pltpu.make_async_copy(
            rows_w_v[p], out_hbm.at[pl.ds(off, CHUNK), pl.ds(0, WORD_DIM)],
            sem_o[p])
        ct = pltpu.make_async_copy(
            rows_t_v[p], out_hbm.at[pl.ds(off, CHUNK), pl.ds(WORD_DIM, TAG_DIM)],
            sem_o[p])
        return cw, ct

    def body(io, _):
        for b in range(NBUF):
            i = io * NBUF + b
            cw, ct = out_copies(i, b)

            @pl.when(io > 0)
            def _wait_prev_out():
                cw.wait()
                ct.wait()

            gather_chunk(i, b)
            tag_chunk(i, b)
            pltpu.make_async_copy(
                word_table_hbm.at[idx_w_v.at[pl.ds(i * CHUNK, CHUNK)]],
                rows_w_v[b], sem_g[b]).wait()
            cw2, ct2 = out_copies(i, b)
            cw2.start()
            ct2.start()
        return ()

    lax.fori_loop(0, n_chunks // NBUF, body, (), unroll=False)

    for b in range(NBUF):
        cw, ct = out_copies(n_chunks - NBUF + b, b)
        cw.wait()
        ct.wait()


def kernel(word_ids, tag_ids, word_table, tag_table):
    b, s = word_ids.shape
    n_total = b * s
    wf = word_ids.reshape(n_total).astype(jnp.int32)
    tf = tag_ids.reshape(n_total).astype(jnp.int32)
    tag_flat = tag_table.reshape(TAG_NUM * TAG_DIM)
    per_w = n_total // NW

    run = pl.kernel(
        _emb_kernel,
        out_type=jax.ShapeDtypeStruct((n_total, OUT_DIM), jnp.float32),
        mesh=plsc.VectorSubcoreMesh(core_axis_name="c", subcore_axis_name="s"),
        scratch_types=[
            pltpu.VMEM((per_w,), jnp.int32),
            pltpu.VMEM((per_w,), jnp.int32),
            pltpu.VMEM((TAG_NUM * TAG_DIM,), jnp.float32),
            [pltpu.VMEM((CHUNK, WORD_DIM), jnp.float32) for _ in range(NBUF)],
            [pltpu.VMEM((CHUNK, TAG_DIM), jnp.float32) for _ in range(NBUF)],
            [pltpu.SemaphoreType.DMA for _ in range(NBUF)],
            [pltpu.SemaphoreType.DMA for _ in range(NBUF)],
        ],
    )
    out = run(wf, tf, word_table, tag_flat)
    return out.reshape(b, s, OUT_DIM)
